# baseline (speedup 1.0000x reference)
import functools

import jax
import jax.numpy as jnp
from jax import lax
from jax.experimental import pallas as pl
from jax.experimental.pallas import tpu as pltpu
from jax.experimental.pallas import tpu_sc as plsc

VOCAB = 1000000
DIM = 64
NUM_BITS = 16
BITS_BEGIN = 8
B = 16384
L = 50

N = B * L
NC, NS = 2, 16
NW = NC * NS
BT = B // 128
NUNIT = L * BT
PER_W = NUNIT // NW
NBUF = 2
GROUPS = PER_W // NBUF
OBLK = DIM * 128


def _make_kernel():
  mesh = plsc.VectorSubcoreMesh(core_axis_name="c", subcore_axis_name="s")

  @functools.partial(
      pl.kernel,
      mesh=mesh,
      compiler_params=pltpu.CompilerParams(
          use_tc_tiling_on_sc=False, needs_layout_passes=False),
      out_type=jax.ShapeDtypeStruct((N * DIM,), jnp.float32),
      scratch_types=(
          [pltpu.VMEM((PER_W, 128), jnp.int32)]
          + [pltpu.VMEM((128, DIM), jnp.float32) for _ in range(NBUF)]
          + [pltpu.VMEM((OBLK,), jnp.float32) for _ in range(NBUF)]
          + [pltpu.SemaphoreType.DMA for _ in range(2 * NBUF)]
      ),
  )
  def embed_kernel(ids_hbm, table_hbm, out_hbm, idx2d, *rest):
    gbufs = rest[:NBUF]
    obufs = rest[NBUF:2 * NBUF]
    gsems = rest[2 * NBUF:3 * NBUF]
    osems = rest[3 * NBUF:]
    wid = lax.axis_index("s") * NC + lax.axis_index("c")
    u_base = wid * PER_W
    lanes = jnp.arange(16, dtype=jnp.int32)

    def start_gather(b, k):
      pltpu.async_copy(table_hbm.at[idx2d.at[k]], gbufs[b], gsems[b])

    def wait_gather(b):
      pltpu.make_async_copy(table_hbm.at[idx2d.at[0]], gbufs[b],
                            gsems[b]).wait()

    def out_base(k):
      u = u_base + k
      l = u // BT
      bt = u - l * BT
      return l * (DIM * B) + bt * 1024

    def start_out(b, base):
      for dt in range(DIM // 8):
        pltpu.async_copy(
            obufs[b].at[pl.ds(dt * 1024, 1024)],
            out_hbm.at[pl.ds(base + dt * (8 * B), 1024)],
            osems[b])

    def wait_out(b):
      pltpu.make_async_copy(obufs[b], out_hbm.at[pl.ds(0, OBLK)],
                            osems[b]).wait()

    def build(b, k):
      def group(g, carry):
        g16 = g * 16
        for j in range(16):
          for q in range(DIM // 16):
            vals = gbufs[b][g16 + j, pl.ds(q * 16, 16)]
            pos = (q * 16 + lanes) * 128 + j + g16
            plsc.store_scatter(obufs[b], [pos], vals)
        ids16 = idx2d[k, pl.ds(g16, 16)]
        for j in range(NUM_BITS):
          bits = ((ids16 >> j) & 1).astype(jnp.float32)
          obufs[b][pl.ds((BITS_BEGIN + j) * 128 + g16, 16)] = bits
        return carry

      lax.fori_loop(0, 8, group, 0)

    pltpu.sync_copy(ids_hbm.at[pl.ds(wid * PER_W, PER_W)], idx2d)
    for b in range(NBUF):
      start_gather(b, b)
      start_out(b, out_base(b))

    def group_body(g, carry):
      for b in range(NBUF):
        k = g * NBUF + b
        wait_gather(b)
        wait_out(b)
        build(b, k)
        start_out(b, out_base(k))

        @pl.when(k + NBUF < PER_W)
        def _():
          start_gather(b, k + NBUF)
      return carry

    lax.fori_loop(0, GROUPS, group_body, 0)
    for b in range(NBUF):
      wait_out(b)

  return embed_kernel


_EMBED = _make_kernel()


@jax.jit
def kernel(token_ids, table):
  ids = token_ids.astype(jnp.int32).T.reshape(NUNIT, 128)
  out = _EMBED(ids, table)
  p5 = out.reshape(L, DIM // 8, BT, 8, 128)
  return jnp.transpose(p5, (2, 4, 0, 1, 3)).reshape(B, L, DIM)


# speedup vs baseline: 1.2519x; 1.2519x over previous
import functools

import jax
import jax.numpy as jnp
from jax import lax
from jax.experimental import pallas as pl
from jax.experimental.pallas import tpu as pltpu
from jax.experimental.pallas import tpu_sc as plsc

VOCAB = 1000000
DIM = 64
NUM_BITS = 16
BITS_BEGIN = 8
B = 16384
L = 50

N = B * L
NC, NS = 2, 16
NW = NC * NS
BT = B // 128
NUNIT = L * BT
PER_W = NUNIT // NW
NBUF = 2
GROUPS = PER_W // NBUF
OBLK = DIM * 128


def _make_kernel():
  mesh = plsc.VectorSubcoreMesh(core_axis_name="c", subcore_axis_name="s")

  @functools.partial(
      pl.kernel,
      mesh=mesh,
      compiler_params=pltpu.CompilerParams(
          use_tc_tiling_on_sc=False, needs_layout_passes=False),
      out_type=jax.ShapeDtypeStruct((N * DIM,), jnp.float32),
      scratch_types=(
          [pltpu.VMEM((PER_W, 128), jnp.int32)]
          + [pltpu.VMEM((128, DIM), jnp.float32) for _ in range(NBUF)]
          + [pltpu.VMEM((OBLK,), jnp.float32) for _ in range(NBUF)]
          + [pltpu.SemaphoreType.DMA for _ in range(2 * NBUF)]
      ),
  )
  def embed_kernel(ids_hbm, table_hbm, out_hbm, idx2d, *rest):
    gbufs = rest[:NBUF]
    obufs = rest[NBUF:2 * NBUF]
    gsems = rest[2 * NBUF:3 * NBUF]
    osems = rest[3 * NBUF:]
    wid = lax.axis_index("s") * NC + lax.axis_index("c")
    u_base = wid * PER_W
    lanes = jnp.arange(16, dtype=jnp.int32)

    def start_gather(b, k):
      pltpu.async_copy(table_hbm.at[idx2d.at[k]], gbufs[b], gsems[b])

    def wait_gather(b):
      pltpu.make_async_copy(table_hbm.at[idx2d.at[0]], gbufs[b],
                            gsems[b]).wait()

    def out_base(k):
      u = u_base + k
      l = u // BT
      bt = u - l * BT
      return l * (DIM * B) + bt * 1024

    def start_out(b, base):
      for dt in range(DIM // 8):
        pltpu.async_copy(
            obufs[b].at[pl.ds(dt * 1024, 1024)],
            out_hbm.at[pl.ds(base + dt * (8 * B), 1024)],
            osems[b])

    def wait_out(b):
      pltpu.make_async_copy(obufs[b], out_hbm.at[pl.ds(0, OBLK)],
                            osems[b]).wait()

    def build(b, k):
      @plsc.parallel_loop(0, 8, 1, unroll=2)
      def group(g):
        g16 = g * 16
        rowv = lanes + g16
        for q in range(DIM // 16):
          for o in range(16):
            dvec = q * 16 + ((lanes + o) % 16)
            keep = (dvec < BITS_BEGIN) | (dvec >= BITS_BEGIN + NUM_BITS)
            vals = plsc.load_gather(gbufs[b], [rowv, dvec])
            plsc.store_scatter(obufs[b], [dvec * 128 + lanes + g16], vals,
                               mask=keep)
        ids16 = idx2d[k, pl.ds(g16, 16)]
        for j in range(NUM_BITS):
          bits = ((ids16 >> j) & 1).astype(jnp.float32)
          obufs[b][pl.ds((BITS_BEGIN + j) * 128 + g16, 16)] = bits

    pltpu.sync_copy(ids_hbm.at[pl.ds(wid * PER_W, PER_W)], idx2d)
    for b in range(NBUF):
      start_gather(b, b)
      start_out(b, out_base(b))

    def group_body(g, carry):
      for b in range(NBUF):
        k = g * NBUF + b
        wait_gather(b)
        wait_out(b)
        build(b, k)
        start_out(b, out_base(k))

        @pl.when(k + NBUF < PER_W)
        def _():
          start_gather(b, k + NBUF)
      return carry

    lax.fori_loop(0, GROUPS, group_body, 0)
    for b in range(NBUF):
      wait_out(b)

  return embed_kernel


_EMBED = _make_kernel()


@jax.jit
def kernel(token_ids, table):
  ids = token_ids.astype(jnp.int32).T.reshape(NUNIT, 128)
  out = _EMBED(ids, table)
  p5 = out.reshape(L, DIM // 8, BT, 8, 128)
  return jnp.transpose(p5, (2, 4, 0, 1, 3)).reshape(B, L, DIM)


# speedup vs baseline: 2.0562x; 1.6424x over previous
import functools

import jax
import jax.numpy as jnp
from jax import lax
from jax.experimental import pallas as pl
from jax.experimental.pallas import tpu as pltpu
from jax.experimental.pallas import tpu_sc as plsc

VOCAB = 1000000
DIM = 64
NUM_BITS = 16
BITS_BEGIN = 8
B = 16384
L = 50

N = B * L
NC, NS = 2, 16
NW = NC * NS
BT = B // 128
NUNIT = L * BT
PER_W = NUNIT // NW
NBUF = 2
GROUPS = PER_W // NBUF
OBLK = DIM * 128


def _make_kernel():
  mesh = plsc.VectorSubcoreMesh(core_axis_name="c", subcore_axis_name="s")

  @functools.partial(
      pl.kernel,
      mesh=mesh,
      compiler_params=pltpu.CompilerParams(
          use_tc_tiling_on_sc=False, needs_layout_passes=False),
      out_type=jax.ShapeDtypeStruct((N * DIM,), jnp.float32),
      scratch_types=(
          [pltpu.VMEM((PER_W, 128), jnp.int32)]
          + [pltpu.VMEM((128, DIM), jnp.float32) for _ in range(NBUF)]
          + [pltpu.VMEM((OBLK,), jnp.float32) for _ in range(NBUF)]
          + [pltpu.SemaphoreType.DMA for _ in range(2 * NBUF)]
      ),
  )
  def embed_kernel(ids_hbm, table_hbm, out_hbm, idx2d, *rest):
    gbufs = rest[:NBUF]
    obufs = rest[NBUF:2 * NBUF]
    gsems = rest[2 * NBUF:3 * NBUF]
    osems = rest[3 * NBUF:]
    wid = lax.axis_index("s") * NC + lax.axis_index("c")
    u_base = wid * PER_W
    lanes = jnp.arange(16, dtype=jnp.int32)

    def start_gather(b, k):
      pltpu.async_copy(table_hbm.at[idx2d.at[k]], gbufs[b], gsems[b])

    def wait_gather(b):
      pltpu.make_async_copy(table_hbm.at[idx2d.at[0]], gbufs[b],
                            gsems[b]).wait()

    def out_base(k):
      u = u_base + k
      l = u // BT
      bt = u - l * BT
      return l * (DIM * B) + bt * 1024

    def start_out(b, base):
      for dt in range(DIM // 8):
        pltpu.async_copy(
            obufs[b].at[pl.ds(dt * 1024, 1024)],
            out_hbm.at[pl.ds(base + dt * (8 * B), 1024)],
            osems[b])

    def wait_out(b):
      pltpu.make_async_copy(obufs[b], out_hbm.at[pl.ds(0, OBLK)],
                            osems[b]).wait()

    def build(b, k):
      @plsc.parallel_loop(0, 8, 1, unroll=4)
      def group(g):
        g16 = g * 16
        rowv = lanes + g16
        for blk in range(3):
          for o in range(16):
            rot = (lanes + o) % 16
            if blk == 0:
              dvec = rot + (rot >= BITS_BEGIN) * 16
            else:
              dvec = rot + 16 * (blk + 1)
            vals = plsc.load_gather(gbufs[b], [rowv, dvec])
            plsc.store_scatter(obufs[b], [dvec * 128 + lanes + g16], vals)
        ids16 = idx2d[k, pl.ds(g16, 16)]
        for j in range(NUM_BITS):
          bits = ((ids16 >> j) & 1).astype(jnp.float32)
          obufs[b][pl.ds((BITS_BEGIN + j) * 128 + g16, 16)] = bits

    pltpu.sync_copy(ids_hbm.at[pl.ds(wid * PER_W, PER_W)], idx2d)
    for b in range(NBUF):
      start_gather(b, b)
      start_out(b, out_base(b))

    def group_body(g, carry):
      for b in range(NBUF):
        k = g * NBUF + b
        wait_gather(b)
        wait_out(b)
        build(b, k)
        start_out(b, out_base(k))

        @pl.when(k + NBUF < PER_W)
        def _():
          start_gather(b, k + NBUF)
      return carry

    lax.fori_loop(0, GROUPS, group_body, 0)
    for b in range(NBUF):
      wait_out(b)

  return embed_kernel


_EMBED = _make_kernel()


@jax.jit
def kernel(token_ids, table):
  ids = token_ids.astype(jnp.int32).T.reshape(NUNIT, 128)
  out = _EMBED(ids, table)
  p5 = out.reshape(L, DIM // 8, BT, 8, 128)
  return jnp.transpose(p5, (2, 4, 0, 1, 3)).reshape(B, L, DIM)
